# Initial kernel scaffold; baseline (speedup 1.0000x reference)
import jax
import jax.numpy as jnp
from jax.experimental import pallas as pl


def kernel(x, edge_index, edge_attr, W1, b1, W2, b2):
    raise NotImplementedError("write your pallas kernel here")


# speedup vs baseline: 2.5633x; 2.5633x over previous
import functools

import jax
import jax.numpy as jnp
from jax import lax
from jax.experimental import pallas as pl
from jax.experimental.pallas import tpu as pltpu
from jax.experimental.pallas import tpu_sc as plsc

NC = 2
NS = 16
NW = NC * NS
CH = 128


def _mesh():
    return plsc.VectorSubcoreMesh(core_axis_name="c", subcore_axis_name="s")


def _npad(n):
    return ((n + CH * NS - 1) // (CH * NS)) * (CH * NS)


def _node_matmuls(x, w1a, w1b):
    n, df = x.shape
    do = w1a.shape[1]
    bn = 2048

    def body(x_ref, wa_ref, wb_ref, p_ref, q_ref):
        xv = x_ref[...]
        p_ref[...] = jnp.dot(xv, wa_ref[...], preferred_element_type=jnp.float32)
        q_ref[...] = jnp.dot(xv, wb_ref[...], preferred_element_type=jnp.float32)

    return pl.pallas_call(
        body,
        grid=(n // bn,),
        in_specs=[
            pl.BlockSpec((bn, df), lambda i: (i, 0)),
            pl.BlockSpec((df, do), lambda i: (0, 0)),
            pl.BlockSpec((df, do), lambda i: (0, 0)),
        ],
        out_specs=[
            pl.BlockSpec((bn, do), lambda i: (i, 0)),
            pl.BlockSpec((bn, do), lambda i: (i, 0)),
        ],
        out_shape=[
            jax.ShapeDtypeStruct((n, do), jnp.float32),
            jax.ShapeDtypeStruct((n, do), jnp.float32),
        ],
    )(x, w1a, w1b)


def _sc_gather(p, q, row, col):
    d = p.shape[1]
    e = row.shape[0]
    epw = e // NW
    nfull = epw // CH

    @functools.partial(
        pl.kernel,
        out_type=(jax.ShapeDtypeStruct((e, d), jnp.float32),
                  jax.ShapeDtypeStruct((e, d), jnp.float32)),
        mesh=_mesh(),
        scratch_types=[
            pltpu.VMEM((CH,), jnp.int32),
            pltpu.VMEM((CH,), jnp.int32),
            pltpu.VMEM((CH, d), jnp.float32),
            pltpu.VMEM((CH, d), jnp.float32),
            pltpu.SemaphoreType.DMA,
            pltpu.SemaphoreType.DMA,
        ],
    )
    def k(p_hbm, q_hbm, row_hbm, col_hbm, gp_hbm, gq_hbm,
          ri, ci, bp, bq, s1, s2):
        wid = lax.axis_index("s") * NC + lax.axis_index("c")
        base = wid * epw

        def body(i, _):
            off = base + i * CH
            pltpu.sync_copy(row_hbm.at[pl.ds(off, CH)], ri)
            pltpu.sync_copy(col_hbm.at[pl.ds(off, CH)], ci)
            c1 = pltpu.async_copy(p_hbm.at[ri], bp, s1)
            c2 = pltpu.async_copy(q_hbm.at[ci], bq, s2)
            c1.wait()
            c2.wait()
            pltpu.sync_copy(bp, gp_hbm.at[pl.ds(off, CH), :])
            pltpu.sync_copy(bq, gq_hbm.at[pl.ds(off, CH), :])
            return 0

        lax.fori_loop(0, nfull, body, 0)

    return k(p, q, row, col)


def _sc_counts(row, npad, d):
    e = row.shape[0]
    epw = e // NW
    nfull = epw // CH
    npc = npad // NS
    nzc = npc // CH

    zeros = jnp.zeros((CH, d), jnp.float32)
    ones = jnp.ones((CH, d), jnp.float32)

    @functools.partial(
        pl.kernel,
        out_type=jax.ShapeDtypeStruct((NC * npad, d), jnp.float32),
        mesh=_mesh(),
        scratch_types=[
            pltpu.VMEM((CH,), jnp.int32),
            pltpu.VMEM((CH, d), jnp.float32),
            pltpu.VMEM((CH, d), jnp.float32),
            pltpu.VMEM_SHARED((npad, d), jnp.float32),
        ],
    )
    def k(row_hbm, z_hbm, on_hbm, cnt_hbm, idx, buf, onev, cnt_s):
        cid = lax.axis_index("c")
        sid = lax.axis_index("s")
        nbase = sid * npc
        pltpu.sync_copy(z_hbm, buf)

        def zbody(j, _):
            pltpu.sync_copy(buf, cnt_s.at[pl.ds(nbase + j * CH, CH), :])
            return 0

        lax.fori_loop(0, nzc, zbody, 0)
        pltpu.sync_copy(on_hbm, onev)
        plsc.subcore_barrier()

        base = (cid * NS + sid) * epw

        def body(i, _):
            pltpu.sync_copy(row_hbm.at[pl.ds(base + i * CH, CH)], idx)
            pltpu.sync_copy(onev, cnt_s.at[idx], add=True)
            return 0

        lax.fori_loop(0, nfull, body, 0)
        plsc.subcore_barrier()
        obase = cid * npad + nbase

        def obody(j, _):
            pltpu.sync_copy(cnt_s.at[pl.ds(nbase + j * CH, CH), :], buf)
            pltpu.sync_copy(buf, cnt_hbm.at[pl.ds(obase + j * CH, CH), :])
            return 0

        lax.fori_loop(0, nzc, obody, 0)

    return k(row, zeros, ones)


def _edge_mlp(gp, gq, ea, w1c, b1, w2, b2):
    e, d = gp.shape
    de = ea.shape[1]
    be = 1280

    def body(gp_ref, gq_ref, ea_ref, w1c_ref, b1_ref, w2_ref, b2_ref, o_ref):
        pre = (gp_ref[...] + gq_ref[...] + b1_ref[...]
               + jnp.dot(ea_ref[...], w1c_ref[...],
                         preferred_element_type=jnp.float32))
        h = jnp.maximum(pre, 0.0)
        o_ref[...] = (jnp.dot(h, w2_ref[...], preferred_element_type=jnp.float32)
                      + b2_ref[...])

    return pl.pallas_call(
        body,
        grid=(e // be,),
        in_specs=[
            pl.BlockSpec((be, d), lambda i: (i, 0)),
            pl.BlockSpec((be, d), lambda i: (i, 0)),
            pl.BlockSpec((be, de), lambda i: (i, 0)),
            pl.BlockSpec((de, d), lambda i: (0, 0)),
            pl.BlockSpec((1, d), lambda i: (0, 0)),
            pl.BlockSpec((d, d), lambda i: (0, 0)),
            pl.BlockSpec((1, d), lambda i: (0, 0)),
        ],
        out_specs=pl.BlockSpec((be, d), lambda i: (i, 0)),
        out_shape=jax.ShapeDtypeStruct((e, d), jnp.float32),
    )(gp, gq, ea, w1c, b1, w2, b2)


def _sc_scatter(edge_out, row, npad):
    e, d = edge_out.shape
    epw = e // NW
    nfull = epw // CH
    npc = npad // NS
    nzc = npc // CH

    zeros = jnp.zeros((CH, d), jnp.float32)

    @functools.partial(
        pl.kernel,
        out_type=jax.ShapeDtypeStruct((NC * npad, d), jnp.float32),
        mesh=_mesh(),
        scratch_types=[
            pltpu.VMEM((CH,), jnp.int32),
            pltpu.VMEM((CH, d), jnp.float32),
            pltpu.VMEM_SHARED((npad, d), jnp.float32),
        ],
    )
    def k(eo_hbm, row_hbm, zs_hbm, sums_hbm, idx, val, acc_s):
        cid = lax.axis_index("c")
        sid = lax.axis_index("s")
        nbase = sid * npc
        pltpu.sync_copy(zs_hbm, val)

        def zbody(j, _):
            pltpu.sync_copy(val, acc_s.at[pl.ds(nbase + j * CH, CH), :])
            return 0

        lax.fori_loop(0, nzc, zbody, 0)
        plsc.subcore_barrier()

        base = (cid * NS + sid) * epw

        def body(i, _):
            off = base + i * CH
            pltpu.sync_copy(row_hbm.at[pl.ds(off, CH)], idx)
            pltpu.sync_copy(eo_hbm.at[pl.ds(off, CH), :], val)
            pltpu.sync_copy(val, acc_s.at[idx], add=True)
            return 0

        lax.fori_loop(0, nfull, body, 0)
        plsc.subcore_barrier()
        obase = cid * npad + nbase

        def obody(j, _):
            pltpu.sync_copy(acc_s.at[pl.ds(nbase + j * CH, CH), :], val)
            pltpu.sync_copy(val, sums_hbm.at[pl.ds(obase + j * CH, CH), :])
            return 0

        lax.fori_loop(0, nzc, obody, 0)

    return k(edge_out, row, zeros)


def _finalize(s0, s1, c0, c1):
    n, d = s0.shape
    cw = c0.shape[1]
    bn = 2000

    def body(s0_ref, s1_ref, c0_ref, c1_ref, o_ref):
        cnt = c0_ref[...][:, 0:1] + c1_ref[...][:, 0:1]
        o_ref[...] = (s0_ref[...] + s1_ref[...]) / jnp.maximum(cnt, 1.0)

    return pl.pallas_call(
        body,
        grid=(n // bn,),
        in_specs=[
            pl.BlockSpec((bn, d), lambda i: (i, 0)),
            pl.BlockSpec((bn, d), lambda i: (i, 0)),
            pl.BlockSpec((bn, cw), lambda i: (i, 0)),
            pl.BlockSpec((bn, cw), lambda i: (i, 0)),
        ],
        out_specs=pl.BlockSpec((bn, d), lambda i: (i, 0)),
        out_shape=jax.ShapeDtypeStruct((n, d), jnp.float32),
    )(s0, s1, c0, c1)


def kernel(x, edge_index, edge_attr, W1, b1, W2, b2):
    n, df = x.shape
    do = W2.shape[1]
    e = edge_index.shape[1]
    de = edge_attr.shape[1]
    row = edge_index[0]
    col = edge_index[1]

    w1a = W1[:df]
    w1b = W1[df:2 * df]
    w1c = W1[2 * df:]

    npad = _npad(n)
    epad = ((e + NW * CH - 1) // (NW * CH)) * (NW * CH)
    pad_e = epad - e
    pad_n = npad - n

    xp = jnp.concatenate([x, jnp.zeros((pad_n, df), x.dtype)], axis=0)
    sac = jnp.full((pad_e,), npad - 1, row.dtype)
    rowp = jnp.concatenate([row, sac])
    colp = jnp.concatenate([col, sac])
    eap = jnp.concatenate([edge_attr, jnp.zeros((pad_e, de), edge_attr.dtype)],
                          axis=0)

    p, q = _node_matmuls(xp, w1a, w1b)
    gp, gq = _sc_gather(p, q, rowp, colp)
    cnts = _sc_counts(rowp, npad, do)
    edge_out_p = _edge_mlp(gp, gq, eap, w1c,
                           b1.reshape(1, do), W2, b2.reshape(1, do))
    sums = _sc_scatter(edge_out_p, rowp, npad)
    node_out = _finalize(sums[:n], sums[npad:npad + n],
                         cnts[:n], cnts[npad:npad + n])
    return (node_out, edge_out_p[:e])
